# baseline (speedup 1.0000x reference)
import functools

import jax
import jax.numpy as jnp
from jax import lax
from jax.experimental import pallas as pl
from jax.experimental.pallas import tpu as pltpu
from jax.experimental.pallas import tpu_sc as plsc

NUM_EMB = 1000000
EMB_DIM = 16
BATCH = 16384
HIST = 20
THD_NEG = -8.0
THD_POS = 7.0

NC = 2
NS = 16
NW = NC * NS

B_PER_W = BATCH // NW
CB = 64
CHUNKS = B_PER_W // CB
IDX_PER_CHUNK = CB * HIST
IDX_ROWS = IDX_PER_CHUNK // 128
ROUND_MAGIC = 12582912.0


def _sc_body(idx_hbm, w_hbm, s_hbm, out_hbm, idx_v, rows_v, out_v, s_v, sem):
    wid = lax.axis_index("s") * NC + lax.axis_index("c")

    pltpu.sync_copy(s_hbm, s_v)
    s_vec = s_v[...]

    pltpu.sync_copy(idx_hbm.at[pl.ds(wid * (CHUNKS * IDX_ROWS), CHUNKS * IDX_ROWS)], idx_v)

    def chunk_body(c, _):
        base_bag = wid * B_PER_W + c * CB

        copies = []
        for j in range(IDX_ROWS):
            copies.append(
                pltpu.async_copy(
                    w_hbm.at[idx_v.at[c * IDX_ROWS + j]],
                    rows_v.at[pl.ds(j * 128, 128)],
                    sem,
                )
            )
        for cp in copies:
            cp.wait()

        def bag_body(b, _):
            r0 = b * HIST
            acc = rows_v[r0, :]
            for h in range(1, HIST):
                acc = acc + rows_v[r0 + h, :]
            x = acc / s_vec
            x = jnp.minimum(jnp.maximum(x, THD_NEG), THD_POS)
            x = (x + ROUND_MAGIC) - ROUND_MAGIC
            out_v[b, :] = x * s_vec
            return 0

        lax.fori_loop(0, CB, bag_body, 0)

        pltpu.sync_copy(out_v, out_hbm.at[pl.ds(base_bag, CB)])
        return 0

    lax.fori_loop(0, CHUNKS, chunk_body, 0)


def kernel(indices, W, s):
    idx2d = indices.reshape(BATCH * HIST // 128, 128)
    s16 = jnp.broadcast_to(s, (16,)).astype(jnp.float32)

    mesh = plsc.VectorSubcoreMesh(core_axis_name="c", subcore_axis_name="s")
    k = functools.partial(
        pl.kernel,
        mesh=mesh,
        compiler_params=pltpu.CompilerParams(use_tc_tiling_on_sc=False),
        out_type=jax.ShapeDtypeStruct((BATCH, EMB_DIM), jnp.float32),
        scratch_types=[
            pltpu.VMEM((CHUNKS * IDX_ROWS, 128), jnp.int32),
            pltpu.VMEM((IDX_PER_CHUNK, EMB_DIM), jnp.float32),
            pltpu.VMEM((CB, EMB_DIM), jnp.float32),
            pltpu.VMEM((16,), jnp.float32),
            pltpu.SemaphoreType.DMA,
        ],
    )(_sc_body)
    return k(idx2d, W, s16)


# speedup vs baseline: 1.0003x; 1.0003x over previous
import functools

import jax
import jax.numpy as jnp
from jax import lax
from jax.experimental import pallas as pl
from jax.experimental.pallas import tpu as pltpu
from jax.experimental.pallas import tpu_sc as plsc

NUM_EMB = 1000000
EMB_DIM = 16
BATCH = 16384
HIST = 20
THD_NEG = -8.0
THD_POS = 7.0

NC = 2
NS = 16
NW = NC * NS

B_PER_W = BATCH // NW
CB = 64
CHUNKS = B_PER_W // CB
IDX_PER_CHUNK = CB * HIST
IDX_ROWS = IDX_PER_CHUNK // 128
IDX_PER_W = B_PER_W * HIST
ROUND_MAGIC = 12582912.0


def _sc_body(idx_hbm, w_hbm, s_hbm, out_hbm, idx_v, rows_v, out_v, s_v, sem):
    wid = lax.axis_index("s") * NC + lax.axis_index("c")

    pltpu.sync_copy(s_hbm, s_v)
    s_vec = s_v[...]

    pltpu.sync_copy(idx_hbm.at[pl.ds(wid * IDX_PER_W, IDX_PER_W)], idx_v)

    def chunk_body(c, _):
        base_out = (wid * B_PER_W + c * CB) * EMB_DIM

        copies = []
        for j in range(IDX_ROWS):
            copies.append(
                pltpu.async_copy(
                    w_hbm.at[idx_v.at[pl.ds((c * IDX_ROWS + j) * 128, 128)]],
                    rows_v.at[pl.ds(j * 128, 128)],
                    sem,
                )
            )
        for cp in copies:
            cp.wait()

        def bag_body(b, _):
            r0 = b * HIST
            acc = rows_v[r0, :]
            for h in range(1, HIST):
                acc = acc + rows_v[r0 + h, :]
            x = acc / s_vec
            x = jnp.minimum(jnp.maximum(x, THD_NEG), THD_POS)
            x = (x + ROUND_MAGIC) - ROUND_MAGIC
            out_v[pl.ds(b * EMB_DIM, EMB_DIM)] = x * s_vec
            return 0

        lax.fori_loop(0, CB, bag_body, 0)

        pltpu.sync_copy(out_v, out_hbm.at[pl.ds(base_out, CB * EMB_DIM)])
        return 0

    lax.fori_loop(0, CHUNKS, chunk_body, 0)


def kernel(indices, W, s):
    idx1d = indices.reshape(BATCH * HIST)
    s16 = jnp.broadcast_to(s, (16,)).astype(jnp.float32)

    mesh = plsc.VectorSubcoreMesh(core_axis_name="c", subcore_axis_name="s")
    k = functools.partial(
        pl.kernel,
        mesh=mesh,
        compiler_params=pltpu.CompilerParams(use_tc_tiling_on_sc=False),
        out_type=jax.ShapeDtypeStruct((BATCH * EMB_DIM,), jnp.float32),
        scratch_types=[
            pltpu.VMEM((IDX_PER_W,), jnp.int32),
            pltpu.VMEM((IDX_PER_CHUNK, EMB_DIM), jnp.float32),
            pltpu.VMEM((CB * EMB_DIM,), jnp.float32),
            pltpu.VMEM((16,), jnp.float32),
            pltpu.SemaphoreType.DMA,
        ],
    )(_sc_body)
    out = k(idx1d, W, s16)
    return out.reshape(BATCH, EMB_DIM)
